# baseline (speedup 1.0000x reference)
import functools

import jax
import jax.numpy as jnp
from jax import lax
from jax.experimental import pallas as pl
from jax.experimental.pallas import tpu as pltpu
from jax.experimental.pallas import tpu_sc as plsc

VOCAB = 1000000
EMBED_DIM = 64
BATCH = 4096
HIST = 200

NUM_CORES = 2
NUM_SUBCORES = 16
NUM_WORKERS = NUM_CORES * NUM_SUBCORES

N = BATCH * HIST
CHUNK = 128
PER_WORKER = N // NUM_WORKERS
CHUNKS_PER_WORKER = PER_WORKER // CHUNK
LANES = 16
SUB = EMBED_DIM // LANES


def _sc_body(cids_hbm, wids_hbm, table_hbm, out_hbm,
             idx_c, idx_w, rows_c, rows_w, sem):
    wid = lax.axis_index("s") * NUM_CORES + lax.axis_index("c")
    row0 = wid * CHUNKS_PER_WORKER

    pltpu.sync_copy(cids_hbm.at[pl.ds(row0, CHUNKS_PER_WORKER)], idx_c)
    pltpu.sync_copy(wids_hbm.at[pl.ds(row0, CHUNKS_PER_WORKER)], idx_w)

    @pl.loop(0, CHUNKS_PER_WORKER)
    def _chunk(j):
        cpy_c = pltpu.async_copy(table_hbm.at[idx_c.at[j]], rows_c, sem)
        cpy_w = pltpu.async_copy(table_hbm.at[idx_w.at[j]], rows_w, sem)
        cpy_c.wait()
        cpy_w.wait()

        @pl.loop(0, CHUNK, unroll=4)
        def _add(r):
            for k in range(SUB):
                sl = pl.ds(k * LANES, LANES)
                rows_c[r, sl] = rows_c[r, sl] + rows_w[r, sl]

        pltpu.sync_copy(rows_c, out_hbm.at[pl.ds((row0 + j) * CHUNK, CHUNK)])


@jax.jit
def _run(cids2, wids2, embeddings):
    mesh = plsc.VectorSubcoreMesh(core_axis_name="c", subcore_axis_name="s")
    fn = pl.kernel(
        _sc_body,
        out_type=jax.ShapeDtypeStruct((N, EMBED_DIM), jnp.float32),
        mesh=mesh,
        scratch_types=[
            pltpu.VMEM((CHUNKS_PER_WORKER, CHUNK), jnp.int32),
            pltpu.VMEM((CHUNKS_PER_WORKER, CHUNK), jnp.int32),
            pltpu.VMEM((CHUNK, EMBED_DIM), jnp.float32),
            pltpu.VMEM((CHUNK, EMBED_DIM), jnp.float32),
            pltpu.SemaphoreType.DMA,
        ],
        compiler_params=pltpu.CompilerParams(use_tc_tiling_on_sc=False),
    )
    return fn(cids2, wids2, embeddings)


def kernel(cids, wids, sids, embeddings):
    del sids
    cids2 = cids.astype(jnp.int32).reshape(N // CHUNK, CHUNK)
    wids2 = wids.astype(jnp.int32).reshape(N // CHUNK, CHUNK)
    out = _run(cids2, wids2, embeddings)
    return out.reshape(BATCH, HIST, EMBED_DIM)


# speedup vs baseline: 1.2011x; 1.2011x over previous
import functools

import jax
import jax.numpy as jnp
from jax import lax
from jax.experimental import pallas as pl
from jax.experimental.pallas import tpu as pltpu
from jax.experimental.pallas import tpu_sc as plsc

VOCAB = 1000000
EMBED_DIM = 64
BATCH = 4096
HIST = 200

NUM_CORES = 2
NUM_SUBCORES = 16
NUM_WORKERS = NUM_CORES * NUM_SUBCORES

N = BATCH * HIST
CHUNK = 128
PER_WORKER = N // NUM_WORKERS
CHUNKS_PER_WORKER = PER_WORKER // CHUNK
LANES = 16
SUB = EMBED_DIM // LANES


def _sc_body(cids_hbm, wids_hbm, table_hbm, out_hbm,
             idx_c, idx_w, rows_c, rows_w, sem):
    wid = lax.axis_index("s") * NUM_CORES + lax.axis_index("c")
    row0 = wid * CHUNKS_PER_WORKER

    pltpu.sync_copy(cids_hbm.at[pl.ds(row0, CHUNKS_PER_WORKER)], idx_c)
    pltpu.sync_copy(wids_hbm.at[pl.ds(row0, CHUNKS_PER_WORKER)], idx_w)

    @pl.loop(0, CHUNKS_PER_WORKER)
    def _chunk(j):
        cpy_c = pltpu.async_copy(table_hbm.at[idx_c.at[j]], rows_c, sem)
        cpy_c.wait()
        cpy_w = pltpu.async_copy(table_hbm.at[idx_w.at[j]], rows_c, sem,
                                 add=True)
        cpy_w.wait()

        pltpu.sync_copy(rows_c, out_hbm.at[pl.ds((row0 + j) * CHUNK, CHUNK)])


@jax.jit
def _run(cids2, wids2, embeddings):
    mesh = plsc.VectorSubcoreMesh(core_axis_name="c", subcore_axis_name="s")
    fn = pl.kernel(
        _sc_body,
        out_type=jax.ShapeDtypeStruct((N, EMBED_DIM), jnp.float32),
        mesh=mesh,
        scratch_types=[
            pltpu.VMEM((CHUNKS_PER_WORKER, CHUNK), jnp.int32),
            pltpu.VMEM((CHUNKS_PER_WORKER, CHUNK), jnp.int32),
            pltpu.VMEM((CHUNK, EMBED_DIM), jnp.float32),
            pltpu.VMEM((CHUNK, EMBED_DIM), jnp.float32),
            pltpu.SemaphoreType.DMA,
        ],
        compiler_params=pltpu.CompilerParams(use_tc_tiling_on_sc=False),
    )
    return fn(cids2, wids2, embeddings)


def kernel(cids, wids, sids, embeddings):
    del sids
    cids2 = cids.astype(jnp.int32).reshape(N // CHUNK, CHUNK)
    wids2 = wids.astype(jnp.int32).reshape(N // CHUNK, CHUNK)
    out = _run(cids2, wids2, embeddings)
    return out.reshape(BATCH, HIST, EMBED_DIM)


# speedup vs baseline: 1.4502x; 1.2073x over previous
import functools

import jax
import jax.numpy as jnp
from jax import lax
from jax.experimental import pallas as pl
from jax.experimental.pallas import tpu as pltpu
from jax.experimental.pallas import tpu_sc as plsc

VOCAB = 1000000
EMBED_DIM = 64
BATCH = 4096
HIST = 200

NUM_CORES = 2
NUM_SUBCORES = 16
NUM_WORKERS = NUM_CORES * NUM_SUBCORES

N = BATCH * HIST
CHUNK = 128
PER_WORKER = N // NUM_WORKERS
CHUNKS_PER_WORKER = PER_WORKER // CHUNK
LANES = 16
SUB = EMBED_DIM // LANES


NBUF = 4


def _sc_body(cids_hbm, wids_hbm, table_hbm, out_hbm,
             idx_c, idx_w, rows, sem_gc, sem_gw, sem_s):
    wid = lax.axis_index("s") * NUM_CORES + lax.axis_index("c")
    row0 = wid * CHUNKS_PER_WORKER

    pltpu.sync_copy(cids_hbm.at[pl.ds(row0, CHUNKS_PER_WORKER)], idx_c)
    pltpu.sync_copy(wids_hbm.at[pl.ds(row0, CHUNKS_PER_WORKER)], idx_w)

    def fire_gc(i, b):
        pltpu.async_copy(table_hbm.at[idx_c.at[i]], rows.at[b], sem_gc.at[b])

    def wait_gc(b):
        pltpu.make_async_copy(table_hbm.at[idx_c.at[0]], rows.at[b],
                              sem_gc.at[b]).wait()

    def fire_gw(i, b):
        pltpu.async_copy(table_hbm.at[idx_w.at[i]], rows.at[b], sem_gw.at[b],
                         add=True)

    def wait_gw(b):
        pltpu.make_async_copy(table_hbm.at[idx_w.at[0]], rows.at[b],
                              sem_gw.at[b]).wait()

    def fire_s(i, b):
        pltpu.async_copy(rows.at[b], out_hbm.at[pl.ds((row0 + i) * CHUNK,
                                                      CHUNK)], sem_s.at[b])

    def wait_s(b):
        pltpu.make_async_copy(rows.at[b], out_hbm.at[pl.ds(0, CHUNK)],
                              sem_s.at[b]).wait()

    @pl.loop(0, CHUNKS_PER_WORKER, step=NBUF)
    def _grp(i0):
        for o in range(NBUF):
            i = i0 + o
            b = o

            @pl.when(i >= NBUF)
            def _():
                wait_s(b)

            fire_gc(i, b)

            b1 = (o - 1) % NBUF

            @pl.when(i >= 1)
            def _():
                wait_gc(b1)
                fire_gw(i - 1, b1)

            b2 = (o - 2) % NBUF

            @pl.when(i >= 2)
            def _():
                wait_gw(b2)
                fire_s(i - 2, b2)

    last = CHUNKS_PER_WORKER - 1
    wait_gc((last) % NBUF)
    fire_gw(last, last % NBUF)
    wait_gw((last - 1) % NBUF)
    fire_s(last - 1, (last - 1) % NBUF)
    wait_gw(last % NBUF)
    fire_s(last, last % NBUF)
    for b in range(NBUF):
        wait_s(b)


@jax.jit
def _run(cids2, wids2, embeddings):
    mesh = plsc.VectorSubcoreMesh(core_axis_name="c", subcore_axis_name="s")
    fn = pl.kernel(
        _sc_body,
        out_type=jax.ShapeDtypeStruct((N, EMBED_DIM), jnp.float32),
        mesh=mesh,
        scratch_types=[
            pltpu.VMEM((CHUNKS_PER_WORKER, CHUNK), jnp.int32),
            pltpu.VMEM((CHUNKS_PER_WORKER, CHUNK), jnp.int32),
            pltpu.VMEM((NBUF, CHUNK, EMBED_DIM), jnp.float32),
            pltpu.SemaphoreType.DMA((NBUF,)),
            pltpu.SemaphoreType.DMA((NBUF,)),
            pltpu.SemaphoreType.DMA((NBUF,)),
        ],
        compiler_params=pltpu.CompilerParams(use_tc_tiling_on_sc=False),
    )
    return fn(cids2, wids2, embeddings)


def kernel(cids, wids, sids, embeddings):
    del sids
    cids2 = cids.astype(jnp.int32).reshape(N // CHUNK, CHUNK)
    wids2 = wids.astype(jnp.int32).reshape(N // CHUNK, CHUNK)
    out = _run(cids2, wids2, embeddings)
    return out.reshape(BATCH, HIST, EMBED_DIM)


# speedup vs baseline: 1.4575x; 1.0050x over previous
import functools

import jax
import jax.numpy as jnp
from jax import lax
from jax.experimental import pallas as pl
from jax.experimental.pallas import tpu as pltpu
from jax.experimental.pallas import tpu_sc as plsc

VOCAB = 1000000
EMBED_DIM = 64
BATCH = 4096
HIST = 200

NUM_CORES = 2
NUM_SUBCORES = 16
NUM_WORKERS = NUM_CORES * NUM_SUBCORES

N = BATCH * HIST
CHUNK = 128
PER_WORKER = N // NUM_WORKERS
CHUNKS_PER_WORKER = PER_WORKER // CHUNK
LANES = 16
SUB = EMBED_DIM // LANES


NBUF = 8
LAG = 2


def _sc_body(cids_hbm, wids_hbm, table_hbm, out_hbm,
             idx_c, idx_w, rows, sem_gc, sem_gw, sem_s):
    wid = lax.axis_index("s") * NUM_CORES + lax.axis_index("c")
    row0 = wid * CHUNKS_PER_WORKER

    pltpu.sync_copy(cids_hbm.at[pl.ds(row0, CHUNKS_PER_WORKER)], idx_c)
    pltpu.sync_copy(wids_hbm.at[pl.ds(row0, CHUNKS_PER_WORKER)], idx_w)

    def fire_gc(i, b):
        pltpu.async_copy(table_hbm.at[idx_c.at[i]], rows.at[b], sem_gc.at[b])

    def wait_gc(b):
        pltpu.make_async_copy(table_hbm.at[idx_c.at[0]], rows.at[b],
                              sem_gc.at[b]).wait()

    def fire_gw(i, b):
        pltpu.async_copy(table_hbm.at[idx_w.at[i]], rows.at[b], sem_gw.at[b],
                         add=True)

    def wait_gw(b):
        pltpu.make_async_copy(table_hbm.at[idx_w.at[0]], rows.at[b],
                              sem_gw.at[b]).wait()

    def fire_s(i, b):
        pltpu.async_copy(rows.at[b], out_hbm.at[pl.ds((row0 + i) * CHUNK,
                                                      CHUNK)], sem_s.at[b])

    def wait_s(b):
        pltpu.make_async_copy(rows.at[b], out_hbm.at[pl.ds(0, CHUNK)],
                              sem_s.at[b]).wait()

    @pl.loop(0, CHUNKS_PER_WORKER, step=NBUF)
    def _grp(i0):
        for o in range(NBUF):
            i = i0 + o
            b = o

            @pl.when(i >= NBUF)
            def _():
                wait_s(b)

            fire_gc(i, b)

            b1 = (o - LAG) % NBUF

            @pl.when(i >= LAG)
            def _():
                wait_gc(b1)
                fire_gw(i - LAG, b1)

            b2 = (o - 2 * LAG) % NBUF

            @pl.when(i >= 2 * LAG)
            def _():
                wait_gw(b2)
                fire_s(i - 2 * LAG, b2)

    for i in range(CHUNKS_PER_WORKER, CHUNKS_PER_WORKER + 2 * LAG):
        j1 = i - LAG
        if 0 <= j1 < CHUNKS_PER_WORKER:
            wait_gc(j1 % NBUF)
            fire_gw(j1, j1 % NBUF)
        j2 = i - 2 * LAG
        if 0 <= j2 < CHUNKS_PER_WORKER:
            wait_gw(j2 % NBUF)
            fire_s(j2, j2 % NBUF)
    for b in range(NBUF):
        wait_s(b)


@jax.jit
def _run(cids2, wids2, embeddings):
    mesh = plsc.VectorSubcoreMesh(core_axis_name="c", subcore_axis_name="s")
    fn = pl.kernel(
        _sc_body,
        out_type=jax.ShapeDtypeStruct((N, EMBED_DIM), jnp.float32),
        mesh=mesh,
        scratch_types=[
            pltpu.VMEM((CHUNKS_PER_WORKER, CHUNK), jnp.int32),
            pltpu.VMEM((CHUNKS_PER_WORKER, CHUNK), jnp.int32),
            pltpu.VMEM((NBUF, CHUNK, EMBED_DIM), jnp.float32),
            pltpu.SemaphoreType.DMA((NBUF,)),
            pltpu.SemaphoreType.DMA((NBUF,)),
            pltpu.SemaphoreType.DMA((NBUF,)),
        ],
        compiler_params=pltpu.CompilerParams(use_tc_tiling_on_sc=False),
    )
    return fn(cids2, wids2, embeddings)


def kernel(cids, wids, sids, embeddings):
    del sids
    cids2 = cids.astype(jnp.int32).reshape(N // CHUNK, CHUNK)
    wids2 = wids.astype(jnp.int32).reshape(N // CHUNK, CHUNK)
    out = _run(cids2, wids2, embeddings)
    return out.reshape(BATCH, HIST, EMBED_DIM)
